# baseline (speedup 1.0000x reference)
import functools

import jax
import jax.numpy as jnp
from jax import lax
from jax.experimental import pallas as pl
from jax.experimental.pallas import tpu as pltpu
from jax.experimental.pallas import tpu_sc as plsc

N = 10000
E = 160000
D = 256
DE = 16
HID = 128
OUT_E = 16

_F32 = jnp.float32
_BF16 = jnp.bfloat16


def _dot3(a, b):
    a_hi = a.astype(_BF16)
    a_lo = (a - a_hi.astype(_F32)).astype(_BF16)
    b_hi = b.astype(_BF16)
    b_lo = (b - b_hi.astype(_F32)).astype(_BF16)
    d = functools.partial(jnp.dot, preferred_element_type=_F32)
    return d(a_hi, b_hi) + (d(a_hi, b_lo) + d(a_lo, b_hi))


def _pack_halves(x):
    w = x.shape[-1] // 2
    lo = jax.lax.bitcast_convert_type(
        x[:, :w].astype(_BF16).astype(_F32), jnp.uint32)
    hi = jax.lax.bitcast_convert_type(
        x[:, w:].astype(_BF16).astype(_F32), jnp.uint32)
    return jax.lax.bitcast_convert_type(hi | (lo >> 16), _F32)


def _unpack_halves(p):
    u = jax.lax.bitcast_convert_type(p, jnp.uint32)
    lo = jax.lax.bitcast_convert_type(u << 16, _F32)
    hi = jax.lax.bitcast_convert_type(u & jnp.uint32(0xFFFF0000), _F32)
    return lo, hi


_NC = 2
_NS = 16
_NW = _NC * _NS

_G_EPT = E // _NW
_G_K = 200
_G_CHUNKS = _G_EPT // _G_K
_EA = 79872
_EB = E - _EA
_SC_EPT = E // _NS
_SC_K = 200
_SC_CHUNKS = _SC_EPT // _SC_K
_NPAD = 10240
_SC_NPT = _NPAD // _NS
_SC_NCHUNK = 64


def _prep1_body(a_ref, w_ref, tcol_ref, trow_ref):
    p = _dot3(a_ref[...], w_ref[...])
    tcol_ref[...] = _pack_halves(p[:, : 2 * D])
    trow_ref[...] = _pack_halves(p[:, 2 * D:])


def _prep1(atom_fea, wcat):
    bn = 2000
    return pl.pallas_call(
        _prep1_body,
        grid=(N // bn,),
        in_specs=[
            pl.BlockSpec((bn, D), lambda i: (i, 0)),
            pl.BlockSpec((D, 4 * D), lambda i: (0, 0)),
        ],
        out_specs=[
            pl.BlockSpec((bn, D), lambda i: (i, 0)),
            pl.BlockSpec((bn, D), lambda i: (i, 0)),
        ],
        out_shape=[
            jax.ShapeDtypeStruct((N, D), _F32),
            jax.ShapeDtypeStruct((N, D), _F32),
        ],
    )(atom_fea, wcat)


def _msg_body(prea_ref, preb_ref, ef_ref, w_ref, b_ref, out_ref):
    ec = _dot3(ef_ref[...], w_ref[...])
    af, as_ = _unpack_halves(prea_ref[...])
    bf_, bs = _unpack_halves(preb_ref[...])
    pa_f = af + bf_ + ec[:, :D] + b_ref[:, :D]
    pa_s = as_ + bs + ec[:, D:] + b_ref[:, D:]
    m = jax.nn.sigmoid(pa_f) * jax.nn.softplus(pa_s)
    out_ref[0] = m[:, : D // 2]
    out_ref[1] = m[:, D // 2:]


def _msg_body_aliased(prea_ref, preb_ref, ef_ref, w_ref, b_ref, dummy_ref,
                      out_ref):
    _msg_body(prea_ref, preb_ref, ef_ref, w_ref, b_ref, out_ref)


def _msg(prea, preb, edge_fea, wef, bfs, ne, blk0, alias_from=None):
    be = 256
    args = [prea, preb, edge_fea, wef, bfs]
    in_specs = [
        pl.BlockSpec((be, D), lambda i: (i, 0)),
        pl.BlockSpec((be, D), lambda i: (i, 0)),
        pl.BlockSpec((be, DE), lambda i: (i + blk0, 0)),
        pl.BlockSpec((DE, 2 * D), lambda i: (0, 0)),
        pl.BlockSpec((1, 2 * D), lambda i: (0, 0)),
    ]
    body = _msg_body
    kwargs = {}
    if alias_from is not None:
        args.append(alias_from)
        in_specs.append(pl.BlockSpec(memory_space=pl.ANY))
        kwargs["input_output_aliases"] = {5: 0}
        body = _msg_body_aliased
    return pl.pallas_call(
        body,
        grid=(ne // be,),
        in_specs=in_specs,
        out_specs=pl.BlockSpec((2, be, D // 2), lambda i: (0, i + blk0, 0)),
        out_shape=jax.ShapeDtypeStruct((2, E, D // 2), _F32),
        **kwargs,
    )(*args)


def _mid_body(a_ref, agg_ref, w_ref, oa_ref, trow2_ref, tcol2_ref):
    oa = a_ref[...] + jnp.concatenate([agg_ref[0], agg_ref[1]], axis=-1)
    oa_ref[...] = oa
    r = _dot3(oa, w_ref[...])
    trow2_ref[...] = r[:, :HID]
    tcol2_ref[...] = r[:, HID:]


def _mid(atom_fea, agg2, w1cat):
    bn = 2000
    return pl.pallas_call(
        _mid_body,
        grid=(N // bn,),
        in_specs=[
            pl.BlockSpec((bn, D), lambda i: (i, 0)),
            pl.BlockSpec((2, bn, D // 2), lambda i: (0, i, 0)),
            pl.BlockSpec((D, 2 * HID), lambda i: (0, 0)),
        ],
        out_specs=[
            pl.BlockSpec((bn, D), lambda i: (i, 0)),
            pl.BlockSpec((bn, HID), lambda i: (i, 0)),
            pl.BlockSpec((bn, HID), lambda i: (i, 0)),
        ],
        out_shape=[
            jax.ShapeDtypeStruct((N, D), _F32),
            jax.ShapeDtypeStruct((N, HID), _F32),
            jax.ShapeDtypeStruct((N, HID), _F32),
        ],
    )(atom_fea, agg2, w1cat)


def _final_body(pre2_ref, ef_ref, w1e_ref, b1_ref, w2_ref, b2_ref, out_ref):
    hp = pre2_ref[...] + _dot3(ef_ref[...], w1e_ref[...]) + b1_ref[...]
    h = hp * jax.nn.sigmoid(hp)
    out_ref[...] = _dot3(h, w2_ref[...]) + b2_ref[...]


def _final(pre2, edge_fea, w1e, b1, w2, b2):
    be = 2000
    return pl.pallas_call(
        _final_body,
        grid=(E // be,),
        in_specs=[
            pl.BlockSpec((be, HID), lambda i: (i, 0)),
            pl.BlockSpec((be, DE), lambda i: (i, 0)),
            pl.BlockSpec((DE, HID), lambda i: (0, 0)),
            pl.BlockSpec((1, HID), lambda i: (0, 0)),
            pl.BlockSpec((HID, OUT_E), lambda i: (0, 0)),
            pl.BlockSpec((1, OUT_E), lambda i: (0, 0)),
        ],
        out_specs=pl.BlockSpec((be, OUT_E), lambda i: (i, 0)),
        out_shape=jax.ShapeDtypeStruct((E, OUT_E), _F32),
    )(pre2, edge_fea, w1e, b1, w2, b2)


def _mesh():
    return plsc.VectorSubcoreMesh(core_axis_name="c", subcore_axis_name="s",
                                  num_cores=_NC, num_subcores=_NS)


def _gather_kernel(width, eoff, ne):
    ept = ne // _NW
    k = 104
    chunks = ept // k
    kt = ept - chunks * k

    @functools.partial(
        pl.kernel, mesh=_mesh(),
        out_type=[
            jax.ShapeDtypeStruct((ne, width), _F32),
            jax.ShapeDtypeStruct((ne, width), _F32),
        ],
        scratch_types=[
            pltpu.VMEM((ept,), jnp.int32),
            pltpu.VMEM((ept,), jnp.int32),
            pltpu.VMEM((2, k, width), _F32),
            pltpu.VMEM((2, k, width), _F32),
            pltpu.VMEM((max(kt, 8), width), _F32),
            pltpu.VMEM((max(kt, 8), width), _F32),
        ] + [pltpu.SemaphoreType.DMA] * 8,
    )
    def gather(tcol_hbm, trow_hbm, col_hbm, row_hbm, prea_hbm, preb_hbm,
               colv, rowv, av, bv, avt, bvt,
               sa0, sa1, sb0, sb1, wa0, wa1, wb0, wb1):
        wid = lax.axis_index("s") * _NC + lax.axis_index("c")
        ebase = wid * ept
        pltpu.sync_copy(col_hbm.at[pl.ds(eoff + ebase, ept)], colv)
        pltpu.sync_copy(row_hbm.at[pl.ds(eoff + ebase, ept)], rowv)
        gsems = ((sa0, sb0), (sa1, sb1))
        wsems = ((wa0, wb0), (wa1, wb1))

        def start(i, p):
            sl = pl.ds(i * k, k)
            return (
                pltpu.async_copy(tcol_hbm.at[colv.at[sl]], av.at[p],
                                 gsems[p][0]),
                pltpu.async_copy(trow_hbm.at[rowv.at[sl]], bv.at[p],
                                 gsems[p][1]),
            )

        pend_g = [start(0, 0), None]
        pend_w = [None, None]
        for i in range(chunks):
            p = i % 2
            q = 1 - p
            if i + 1 < chunks:
                if pend_w[q] is not None:
                    pend_w[q][0].wait()
                    pend_w[q][1].wait()
                    pend_w[q] = None
                pend_g[q] = start(i + 1, q)
            pend_g[p][0].wait()
            pend_g[p][1].wait()
            base = ebase + i * k
            pend_w[p] = (
                pltpu.async_copy(av.at[p], prea_hbm.at[pl.ds(base, k)],
                                 wsems[p][0]),
                pltpu.async_copy(bv.at[p], preb_hbm.at[pl.ds(base, k)],
                                 wsems[p][1]),
            )
        if kt:
            tsl = pl.ds(chunks * k, kt)
            ta = pltpu.async_copy(tcol_hbm.at[colv.at[tsl]], avt, sa0)
            tb = pltpu.async_copy(trow_hbm.at[rowv.at[tsl]], bvt, sb0)
            ta.wait()
            tb.wait()
            tbase = ebase + chunks * k
            pend_w.append((
                pltpu.async_copy(avt, prea_hbm.at[pl.ds(tbase, kt)], wa0),
                pltpu.async_copy(bvt, preb_hbm.at[pl.ds(tbase, kt)], wb0),
            ))
        for pw in pend_w:
            if pw is not None:
                pw[0].wait()
                pw[1].wait()

    return gather


def _gather_add_kernel(width):
    k = _G_K
    nvec = width // 16

    @functools.partial(
        pl.kernel, mesh=_mesh(),
        out_type=jax.ShapeDtypeStruct((E, width), _F32),
        scratch_types=[
            pltpu.VMEM((k,), jnp.int32),
            pltpu.VMEM((k,), jnp.int32),
            pltpu.VMEM((k, width), _F32),
            pltpu.VMEM((k, width), _F32),
            pltpu.SemaphoreType.DMA,
            pltpu.SemaphoreType.DMA,
        ],
    )
    def gather(tcol_hbm, trow_hbm, col_hbm, row_hbm, pre_hbm,
               colv, rowv, av, bv, sema, semb):
        wid = lax.axis_index("s") * _NC + lax.axis_index("c")

        def chunk(i, carry):
            base = wid * _G_EPT + i * k
            pltpu.sync_copy(col_hbm.at[pl.ds(base, k)], colv)
            pltpu.sync_copy(row_hbm.at[pl.ds(base, k)], rowv)
            cpa = pltpu.async_copy(tcol_hbm.at[colv], av, sema)
            cpb = pltpu.async_copy(trow_hbm.at[rowv], bv, semb)
            cpa.wait()
            cpb.wait()

            def add_row(j, c2):
                for kk in range(nvec):
                    sl = pl.ds(kk * 16, 16)
                    plsc.addupdate(av.at[j, sl], bv[j, sl])
                return c2

            lax.fori_loop(0, k, add_row, 0, unroll=False)
            pltpu.sync_copy(av, pre_hbm.at[pl.ds(base, k)])
            return carry

        lax.fori_loop(0, _G_CHUNKS, chunk, 0, unroll=False)

    return gather


def _scatter_add(msg_flat, colidx):
    width = D // 2

    @functools.partial(
        pl.kernel, mesh=_mesh(),
        out_type=jax.ShapeDtypeStruct((2 * _NPAD, width), _F32),
        scratch_types=[
            pltpu.VMEM_SHARED((_NPAD, width), _F32),
            pltpu.VMEM((_SC_NCHUNK, width), _F32),
            pltpu.VMEM((_SC_K,), jnp.int32),
            pltpu.VMEM((_SC_K, width), _F32),
        ],
    )
    def scatter(msg_hbm, col_hbm, out_hbm, aggs, stage, colv, mv):
        c = lax.axis_index("c")
        s = lax.axis_index("s")

        def zrow(j, carry):
            for kk in range(width // 16):
                stage[j, pl.ds(kk * 16, 16)] = jnp.zeros((16,), _F32)
            return carry

        lax.fori_loop(0, _SC_NCHUNK, zrow, 0, unroll=False)

        def zcopy(t, carry):
            pltpu.sync_copy(
                stage, aggs.at[pl.ds(s * _SC_NPT + t * _SC_NCHUNK,
                                     _SC_NCHUNK)])
            return carry

        lax.fori_loop(0, _SC_NPT // _SC_NCHUNK, zcopy, 0, unroll=False)
        plsc.subcore_barrier()

        def chunk(i, carry):
            base = s * _SC_EPT + i * _SC_K
            pltpu.sync_copy(col_hbm.at[pl.ds(base, _SC_K)], colv)
            pltpu.sync_copy(msg_hbm.at[pl.ds(c * E + base, _SC_K)], mv)
            pltpu.sync_copy(mv, aggs.at[colv], add=True)
            return carry

        lax.fori_loop(0, _SC_CHUNKS, chunk, 0, unroll=False)
        plsc.subcore_barrier()

        def drain(t, carry):
            nbase = s * _SC_NPT + t * _SC_NCHUNK
            pltpu.sync_copy(aggs.at[pl.ds(nbase, _SC_NCHUNK)], stage)
            pltpu.sync_copy(stage, out_hbm.at[pl.ds(c * _NPAD + nbase,
                                                    _SC_NCHUNK)])
            return carry

        lax.fori_loop(0, _SC_NPT // _SC_NCHUNK, drain, 0, unroll=False)

    return scatter(msg_flat, colidx)


def kernel(atom_fea, edge_idx, edge_fea, batch, distance,
           Wf, bf, Ws, bs, W1, b1, W2, b2):
    row = edge_idx[0].astype(jnp.int32)
    col = edge_idx[1].astype(jnp.int32)

    wcat = jnp.concatenate(
        [Wf[:D], Ws[:D], Wf[D:2 * D], Ws[D:2 * D]], axis=1)
    wef = jnp.concatenate([Wf[2 * D:], Ws[2 * D:]], axis=1)
    bfs = jnp.concatenate([bf, bs])[None, :]
    w1cat = jnp.concatenate([W1[:D], W1[D:2 * D]], axis=1)
    w1e = W1[2 * D:]

    tcol, trow = _prep1(atom_fea, wcat)
    prea_a, preb_a = _gather_kernel(D, 0, _EA)(tcol, trow, col, row)
    prea_b, preb_b = _gather_kernel(D, _EA, _EB)(tcol, trow, col, row)
    msg_a = _msg(prea_a, preb_a, edge_fea, wef, bfs, _EA, 0)
    msg2 = _msg(prea_b, preb_b, edge_fea, wef, bfs, _EB, _EA // 256,
                alias_from=msg_a)
    agg_flat = _scatter_add(msg2.reshape(2 * E, D // 2), col)
    agg2 = agg_flat.reshape(2, _NPAD, D // 2)

    out_atom, trow2, tcol2 = _mid(atom_fea, agg2, w1cat)
    pre2 = _gather_add_kernel(HID)(trow2, tcol2, row, col)
    out_edge = _final(pre2, edge_fea, w1e, b1[None, :], W2, b2[None, :])
    return (out_atom, out_edge)


# speedup vs baseline: 1.2078x; 1.2078x over previous
import functools

import jax
import jax.numpy as jnp
from jax import lax
from jax.experimental import pallas as pl
from jax.experimental.pallas import tpu as pltpu
from jax.experimental.pallas import tpu_sc as plsc

N = 10000
E = 160000
D = 256
DE = 16
HID = 128
OUT_E = 16

_F32 = jnp.float32
_BF16 = jnp.bfloat16


def _dot3(a, b):
    a_hi = a.astype(_BF16)
    a_lo = (a - a_hi.astype(_F32)).astype(_BF16)
    b_hi = b.astype(_BF16)
    b_lo = (b - b_hi.astype(_F32)).astype(_BF16)
    d = functools.partial(jnp.dot, preferred_element_type=_F32)
    return d(a_hi, b_hi) + (d(a_hi, b_lo) + d(a_lo, b_hi))


def _pack_halves(x):
    w = x.shape[-1] // 2
    lo = jax.lax.bitcast_convert_type(
        x[:, :w].astype(_BF16).astype(_F32), jnp.uint32)
    hi = jax.lax.bitcast_convert_type(
        x[:, w:].astype(_BF16).astype(_F32), jnp.uint32)
    return jax.lax.bitcast_convert_type(hi | (lo >> 16), _F32)


def _unpack_halves(p):
    u = jax.lax.bitcast_convert_type(p, jnp.uint32)
    lo = jax.lax.bitcast_convert_type(u << 16, _F32)
    hi = jax.lax.bitcast_convert_type(u & jnp.uint32(0xFFFF0000), _F32)
    return lo, hi


_NC = 2
_NS = 16
_NW = _NC * _NS

_G_EPT = E // _NW
_G_K = 200
_G_CHUNKS = _G_EPT // _G_K
_EA = 79872
_EB = E - _EA
_SC_EPT = E // _NS
_SC_K = 200
_SC_CHUNKS = _SC_EPT // _SC_K
_NPAD = 10240
_SC_NPT = _NPAD // _NS
_SC_NCHUNK = 64


def _prep1_body(a_ref, w_ref, tcol_ref, trow_ref):
    p = _dot3(a_ref[...], w_ref[...])
    tcol_ref[...] = _pack_halves(p[:, : 2 * D])
    trow_ref[...] = _pack_halves(p[:, 2 * D:])


def _prep1(atom_fea, wcat):
    bn = 2000
    return pl.pallas_call(
        _prep1_body,
        grid=(N // bn,),
        in_specs=[
            pl.BlockSpec((bn, D), lambda i: (i, 0)),
            pl.BlockSpec((D, 4 * D), lambda i: (0, 0)),
        ],
        out_specs=[
            pl.BlockSpec((bn, D), lambda i: (i, 0)),
            pl.BlockSpec((bn, D), lambda i: (i, 0)),
        ],
        out_shape=[
            jax.ShapeDtypeStruct((N, D), _F32),
            jax.ShapeDtypeStruct((N, D), _F32),
        ],
    )(atom_fea, wcat)


def _msg_body(prea_ref, preb_ref, ef_ref, w_ref, b_ref, out_ref):
    ec = _dot3(ef_ref[...], w_ref[...])
    af, as_ = _unpack_halves(prea_ref[...])
    bf_, bs = _unpack_halves(preb_ref[...])
    pa_f = af + bf_ + ec[:, :D] + b_ref[:, :D]
    pa_s = as_ + bs + ec[:, D:] + b_ref[:, D:]
    m = jax.nn.sigmoid(pa_f) * jax.nn.softplus(pa_s)
    out_ref[0] = m[:, : D // 2]
    out_ref[1] = m[:, D // 2:]


def _msg(prea, preb, edge_fea, wef, bfs):
    be = 2000
    return pl.pallas_call(
        _msg_body,
        grid=(E // be,),
        in_specs=[
            pl.BlockSpec((be, D), lambda i: (i, 0)),
            pl.BlockSpec((be, D), lambda i: (i, 0)),
            pl.BlockSpec((be, DE), lambda i: (i, 0)),
            pl.BlockSpec((DE, 2 * D), lambda i: (0, 0)),
            pl.BlockSpec((1, 2 * D), lambda i: (0, 0)),
        ],
        out_specs=pl.BlockSpec((2, be, D // 2), lambda i: (0, i, 0)),
        out_shape=jax.ShapeDtypeStruct((2, E, D // 2), _F32),
    )(prea, preb, edge_fea, wef, bfs)


def _mid_body(a_ref, agg_ref, w_ref, oa_ref, trow2_ref, tcol2_ref):
    oa = a_ref[...] + jnp.concatenate([agg_ref[0], agg_ref[1]], axis=-1)
    oa_ref[...] = oa
    r = _dot3(oa, w_ref[...])
    trow2_ref[...] = r[:, :HID]
    tcol2_ref[...] = r[:, HID:]


def _mid(atom_fea, agg2, w1cat):
    bn = 2000
    return pl.pallas_call(
        _mid_body,
        grid=(N // bn,),
        in_specs=[
            pl.BlockSpec((bn, D), lambda i: (i, 0)),
            pl.BlockSpec((2, bn, D // 2), lambda i: (0, i, 0)),
            pl.BlockSpec((D, 2 * HID), lambda i: (0, 0)),
        ],
        out_specs=[
            pl.BlockSpec((bn, D), lambda i: (i, 0)),
            pl.BlockSpec((bn, HID), lambda i: (i, 0)),
            pl.BlockSpec((bn, HID), lambda i: (i, 0)),
        ],
        out_shape=[
            jax.ShapeDtypeStruct((N, D), _F32),
            jax.ShapeDtypeStruct((N, HID), _F32),
            jax.ShapeDtypeStruct((N, HID), _F32),
        ],
    )(atom_fea, agg2, w1cat)


def _final_body(pre2_ref, ef_ref, w1e_ref, b1_ref, w2_ref, b2_ref, out_ref):
    hp = pre2_ref[...] + _dot3(ef_ref[...], w1e_ref[...]) + b1_ref[...]
    h = hp * jax.nn.sigmoid(hp)
    out_ref[...] = _dot3(h, w2_ref[...]) + b2_ref[...]


def _final(pre2, edge_fea, w1e, b1, w2, b2):
    be = 2000
    return pl.pallas_call(
        _final_body,
        grid=(E // be,),
        in_specs=[
            pl.BlockSpec((be, HID), lambda i: (i, 0)),
            pl.BlockSpec((be, DE), lambda i: (i, 0)),
            pl.BlockSpec((DE, HID), lambda i: (0, 0)),
            pl.BlockSpec((1, HID), lambda i: (0, 0)),
            pl.BlockSpec((HID, OUT_E), lambda i: (0, 0)),
            pl.BlockSpec((1, OUT_E), lambda i: (0, 0)),
        ],
        out_specs=pl.BlockSpec((be, OUT_E), lambda i: (i, 0)),
        out_shape=jax.ShapeDtypeStruct((E, OUT_E), _F32),
    )(pre2, edge_fea, w1e, b1, w2, b2)


def _mesh():
    return plsc.VectorSubcoreMesh(core_axis_name="c", subcore_axis_name="s",
                                  num_cores=_NC, num_subcores=_NS)


def _gather_kernel(width, eoff, ne):
    ept = ne // _NW
    k = 104
    chunks = ept // k
    kt = ept - chunks * k

    @functools.partial(
        pl.kernel, mesh=_mesh(),
        out_type=[
            jax.ShapeDtypeStruct((ne, width), _F32),
            jax.ShapeDtypeStruct((ne, width), _F32),
        ],
        scratch_types=[
            pltpu.VMEM((ept,), jnp.int32),
            pltpu.VMEM((ept,), jnp.int32),
            pltpu.VMEM((2, k, width), _F32),
            pltpu.VMEM((2, k, width), _F32),
            pltpu.VMEM((max(kt, 8), width), _F32),
            pltpu.VMEM((max(kt, 8), width), _F32),
        ] + [pltpu.SemaphoreType.DMA] * 8,
    )
    def gather(tcol_hbm, trow_hbm, col_hbm, row_hbm, prea_hbm, preb_hbm,
               colv, rowv, av, bv, avt, bvt,
               sa0, sa1, sb0, sb1, wa0, wa1, wb0, wb1):
        wid = lax.axis_index("s") * _NC + lax.axis_index("c")
        ebase = wid * ept
        pltpu.sync_copy(col_hbm.at[pl.ds(eoff + ebase, ept)], colv)
        pltpu.sync_copy(row_hbm.at[pl.ds(eoff + ebase, ept)], rowv)
        gsems = ((sa0, sb0), (sa1, sb1))
        wsems = ((wa0, wb0), (wa1, wb1))

        def start(i, p):
            sl = pl.ds(i * k, k)
            return (
                pltpu.async_copy(tcol_hbm.at[colv.at[sl]], av.at[p],
                                 gsems[p][0]),
                pltpu.async_copy(trow_hbm.at[rowv.at[sl]], bv.at[p],
                                 gsems[p][1]),
            )

        pend_g = [start(0, 0), None]
        pend_w = [None, None]
        for i in range(chunks):
            p = i % 2
            q = 1 - p
            if i + 1 < chunks:
                if pend_w[q] is not None:
                    pend_w[q][0].wait()
                    pend_w[q][1].wait()
                    pend_w[q] = None
                pend_g[q] = start(i + 1, q)
            pend_g[p][0].wait()
            pend_g[p][1].wait()
            base = ebase + i * k
            pend_w[p] = (
                pltpu.async_copy(av.at[p], prea_hbm.at[pl.ds(base, k)],
                                 wsems[p][0]),
                pltpu.async_copy(bv.at[p], preb_hbm.at[pl.ds(base, k)],
                                 wsems[p][1]),
            )
        if kt:
            tsl = pl.ds(chunks * k, kt)
            ta = pltpu.async_copy(tcol_hbm.at[colv.at[tsl]], avt, sa0)
            tb = pltpu.async_copy(trow_hbm.at[rowv.at[tsl]], bvt, sb0)
            ta.wait()
            tb.wait()
            tbase = ebase + chunks * k
            pend_w.append((
                pltpu.async_copy(avt, prea_hbm.at[pl.ds(tbase, kt)], wa0),
                pltpu.async_copy(bvt, preb_hbm.at[pl.ds(tbase, kt)], wb0),
            ))
        for pw in pend_w:
            if pw is not None:
                pw[0].wait()
                pw[1].wait()

    return gather


def _gather_add_kernel(width):
    k = _G_K
    nvec = width // 16

    @functools.partial(
        pl.kernel, mesh=_mesh(),
        out_type=jax.ShapeDtypeStruct((E, width), _F32),
        scratch_types=[
            pltpu.VMEM((k,), jnp.int32),
            pltpu.VMEM((k,), jnp.int32),
            pltpu.VMEM((k, width), _F32),
            pltpu.VMEM((k, width), _F32),
            pltpu.SemaphoreType.DMA,
            pltpu.SemaphoreType.DMA,
        ],
    )
    def gather(tcol_hbm, trow_hbm, col_hbm, row_hbm, pre_hbm,
               colv, rowv, av, bv, sema, semb):
        wid = lax.axis_index("s") * _NC + lax.axis_index("c")

        def chunk(i, carry):
            base = wid * _G_EPT + i * k
            pltpu.sync_copy(col_hbm.at[pl.ds(base, k)], colv)
            pltpu.sync_copy(row_hbm.at[pl.ds(base, k)], rowv)
            cpa = pltpu.async_copy(tcol_hbm.at[colv], av, sema)
            cpb = pltpu.async_copy(trow_hbm.at[rowv], bv, semb)
            cpa.wait()
            cpb.wait()

            def add_row(j, c2):
                for kk in range(nvec):
                    sl = pl.ds(kk * 16, 16)
                    plsc.addupdate(av.at[j, sl], bv[j, sl])
                return c2

            lax.fori_loop(0, k, add_row, 0, unroll=False)
            pltpu.sync_copy(av, pre_hbm.at[pl.ds(base, k)])
            return carry

        lax.fori_loop(0, _G_CHUNKS, chunk, 0, unroll=False)

    return gather


def _scatter_add(msg_flat, colidx):
    width = D // 2

    @functools.partial(
        pl.kernel, mesh=_mesh(),
        out_type=jax.ShapeDtypeStruct((2 * _NPAD, width), _F32),
        scratch_types=[
            pltpu.VMEM_SHARED((_NPAD, width), _F32),
            pltpu.VMEM((_SC_NCHUNK, width), _F32),
            pltpu.VMEM((_SC_K,), jnp.int32),
            pltpu.VMEM((_SC_K, width), _F32),
        ],
    )
    def scatter(msg_hbm, col_hbm, out_hbm, aggs, stage, colv, mv):
        c = lax.axis_index("c")
        s = lax.axis_index("s")

        def zrow(j, carry):
            for kk in range(width // 16):
                stage[j, pl.ds(kk * 16, 16)] = jnp.zeros((16,), _F32)
            return carry

        lax.fori_loop(0, _SC_NCHUNK, zrow, 0, unroll=False)

        def zcopy(t, carry):
            pltpu.sync_copy(
                stage, aggs.at[pl.ds(s * _SC_NPT + t * _SC_NCHUNK,
                                     _SC_NCHUNK)])
            return carry

        lax.fori_loop(0, _SC_NPT // _SC_NCHUNK, zcopy, 0, unroll=False)
        plsc.subcore_barrier()

        def chunk(i, carry):
            base = s * _SC_EPT + i * _SC_K
            pltpu.sync_copy(col_hbm.at[pl.ds(base, _SC_K)], colv)
            pltpu.sync_copy(msg_hbm.at[pl.ds(c * E + base, _SC_K)], mv)
            pltpu.sync_copy(mv, aggs.at[colv], add=True)
            return carry

        lax.fori_loop(0, _SC_CHUNKS, chunk, 0, unroll=False)
        plsc.subcore_barrier()

        def drain(t, carry):
            nbase = s * _SC_NPT + t * _SC_NCHUNK
            pltpu.sync_copy(aggs.at[pl.ds(nbase, _SC_NCHUNK)], stage)
            pltpu.sync_copy(stage, out_hbm.at[pl.ds(c * _NPAD + nbase,
                                                    _SC_NCHUNK)])
            return carry

        lax.fori_loop(0, _SC_NPT // _SC_NCHUNK, drain, 0, unroll=False)

    return scatter(msg_flat, colidx)


def kernel(atom_fea, edge_idx, edge_fea, batch, distance,
           Wf, bf, Ws, bs, W1, b1, W2, b2):
    row = edge_idx[0].astype(jnp.int32)
    col = edge_idx[1].astype(jnp.int32)

    wcat = jnp.concatenate(
        [Wf[:D], Ws[:D], Wf[D:2 * D], Ws[D:2 * D]], axis=1)
    wef = jnp.concatenate([Wf[2 * D:], Ws[2 * D:]], axis=1)
    bfs = jnp.concatenate([bf, bs])[None, :]
    w1cat = jnp.concatenate([W1[:D], W1[D:2 * D]], axis=1)
    w1e = W1[2 * D:]

    tcol, trow = _prep1(atom_fea, wcat)
    prea, preb = _gather_kernel(D, 0, E)(tcol, trow, col, row)
    msg2 = _msg(prea, preb, edge_fea, wef, bfs)
    agg_flat = _scatter_add(msg2.reshape(2 * E, D // 2), col)
    agg2 = agg_flat.reshape(2, _NPAD, D // 2)

    out_atom, trow2, tcol2 = _mid(atom_fea, agg2, w1cat)
    pre2 = _gather_add_kernel(HID)(trow2, tcol2, row, col)
    out_edge = _final(pre2, edge_fea, w1e, b1[None, :], W2, b2[None, :])
    return (out_atom, out_edge)
